# baseline (speedup 1.0000x reference)
import jax
import jax.numpy as jnp
from jax.experimental import pallas as pl
from jax.experimental.pallas import tpu as pltpu

_SEQ_LEN = 200


def _add_emb_kernel(idx_ref, x_ref, t_ref, o_ref):
    i = idx_ref[pl.program_id(0)]
    row = t_ref[pl.ds(i, 1), :]
    norm = jnp.sqrt(jnp.sum(row * row))
    scale = jnp.where(norm > 1.0, 1.0 / (norm + 1e-7), 1.0)
    o_ref[...] = x_ref[...] + row * scale


def kernel(x, num_people, table):
    total, b, d = x.shape
    blk = 100
    n_blocks = total // blk
    idx = jnp.minimum((jnp.arange(n_blocks, dtype=jnp.int32) * blk) // _SEQ_LEN,
                      jnp.asarray(num_people, jnp.int32) - 1)
    grid_spec = pltpu.PrefetchScalarGridSpec(
        num_scalar_prefetch=1,
        grid=(n_blocks,),
        in_specs=[
            pl.BlockSpec((blk, b, d), lambda p, idx_ref: (p, 0, 0)),
            pl.BlockSpec(table.shape, lambda p, idx_ref: (0, 0)),
        ],
        out_specs=pl.BlockSpec((blk, b, d), lambda p, idx_ref: (p, 0, 0)),
    )
    return pl.pallas_call(
        _add_emb_kernel,
        grid_spec=grid_spec,
        out_shape=jax.ShapeDtypeStruct(x.shape, x.dtype),
    )(idx, x, table)


# speedup vs baseline: 1.4755x; 1.4755x over previous
import functools

import jax
import jax.numpy as jnp
from jax.experimental import pallas as pl
from jax.experimental.pallas import tpu as pltpu

_SEQ_LEN = 200
_GROUPS_PER_BLOCK = 2


def _add_emb_kernel(idx_ref, x_ref, t_ref, o_ref, *, groups):
    p = pl.program_id(0)
    for h in range(groups):
        i = idx_ref[p * groups + h]
        row = t_ref[pl.ds(i, 1), :]
        norm = jnp.sqrt(jnp.sum(row * row))
        scale = jnp.where(norm > 1.0, 1.0 / (norm + 1e-7), 1.0)
        sl = pl.ds(h * _SEQ_LEN, _SEQ_LEN)
        o_ref[sl] = x_ref[sl] + row * scale


def kernel(x, num_people, table):
    total, b, d = x.shape
    blk = _SEQ_LEN * _GROUPS_PER_BLOCK
    n_blocks = total // blk
    n_groups = total // _SEQ_LEN
    idx = jnp.minimum(jnp.arange(n_groups, dtype=jnp.int32),
                      jnp.asarray(num_people, jnp.int32) - 1)
    grid_spec = pltpu.PrefetchScalarGridSpec(
        num_scalar_prefetch=1,
        grid=(n_blocks,),
        in_specs=[
            pl.BlockSpec((blk, b, d), lambda p, idx_ref: (p, 0, 0)),
            pl.BlockSpec(table.shape, lambda p, idx_ref: (0, 0)),
        ],
        out_specs=pl.BlockSpec((blk, b, d), lambda p, idx_ref: (p, 0, 0)),
    )
    return pl.pallas_call(
        functools.partial(_add_emb_kernel, groups=_GROUPS_PER_BLOCK),
        grid_spec=grid_spec,
        out_shape=jax.ShapeDtypeStruct(x.shape, x.dtype),
    )(idx, x, table)


# speedup vs baseline: 1.5113x; 1.0242x over previous
import functools

import jax
import jax.numpy as jnp
from jax.experimental import pallas as pl
from jax.experimental.pallas import tpu as pltpu

_SEQ_LEN = 200
_GROUPS_PER_BLOCK = 5


def _add_emb_kernel(idx_ref, x_ref, t_ref, o_ref, *, groups):
    p = pl.program_id(0)
    for h in range(groups):
        i = idx_ref[p * groups + h]
        row = t_ref[pl.ds(i, 1), :]
        norm = jnp.sqrt(jnp.sum(row * row))
        scale = jnp.where(norm > 1.0, 1.0 / (norm + 1e-7), 1.0)
        sl = pl.ds(h * _SEQ_LEN, _SEQ_LEN)
        o_ref[sl] = x_ref[sl] + row * scale


def kernel(x, num_people, table):
    total, b, d = x.shape
    blk = _SEQ_LEN * _GROUPS_PER_BLOCK
    n_blocks = total // blk
    n_groups = total // _SEQ_LEN
    idx = jnp.minimum(jnp.arange(n_groups, dtype=jnp.int32),
                      jnp.asarray(num_people, jnp.int32) - 1)
    grid_spec = pltpu.PrefetchScalarGridSpec(
        num_scalar_prefetch=1,
        grid=(n_blocks,),
        in_specs=[
            pl.BlockSpec((blk, b, d), lambda p, idx_ref: (p, 0, 0)),
            pl.BlockSpec(table.shape, lambda p, idx_ref: (0, 0)),
        ],
        out_specs=pl.BlockSpec((blk, b, d), lambda p, idx_ref: (p, 0, 0)),
    )
    return pl.pallas_call(
        functools.partial(_add_emb_kernel, groups=_GROUPS_PER_BLOCK),
        grid_spec=grid_spec,
        out_shape=jax.ShapeDtypeStruct(x.shape, x.dtype),
    )(idx, x, table)
